# baseline (speedup 1.0000x reference)
import functools

import jax
import jax.numpy as jnp
from jax import lax
from jax.experimental import pallas as pl
from jax.experimental.pallas import tpu as pltpu
from jax.experimental.pallas import tpu_sc as plsc

_NW = 32
_CHUNK = 400
_NBUF = 2
_L = 16


@functools.partial(jax.jit, static_argnames=("n_rows", "d"))
def _sc_embed(table, idx_flat, n_rows, d):
    b_total = idx_flat.shape[0]
    b_per_w = b_total // _NW
    n_chunks = b_per_w // _CHUNK
    n_groups = _CHUNK // _L
    mesh = plsc.VectorSubcoreMesh(core_axis_name="c", subcore_axis_name="s")

    @functools.partial(
        pl.kernel,
        mesh=mesh,
        out_type=jax.ShapeDtypeStruct((b_total, d), jnp.float32),
        scratch_types=[
            pltpu.VMEM((n_rows * d,), jnp.float32),
            pltpu.VMEM((_CHUNK,), jnp.int32),
            pltpu.VMEM((_CHUNK,), jnp.int32),
            pltpu.VMEM((_CHUNK, d), jnp.float32),
            pltpu.VMEM((_CHUNK, d), jnp.float32),
            pltpu.SemaphoreType.DMA,
            pltpu.SemaphoreType.DMA,
            pltpu.SemaphoreType.DMA,
            pltpu.SemaphoreType.DMA,
            pltpu.SemaphoreType.DMA,
        ],
        compiler_params=pltpu.CompilerParams(needs_layout_passes=False),
    )
    def k(table_hbm, idx_hbm, out_hbm, table_v, idx_v0, idx_v1,
          rows_v0, rows_v1, sem_t, sem_i0, sem_i1, sem_o0, sem_o1):
        cid = lax.axis_index("c")
        sid = lax.axis_index("s")
        wid = sid * 2 + cid
        base0 = wid * b_per_w
        idx_v = (idx_v0, idx_v1)
        rows_v = (rows_v0, rows_v1)
        sem_i = (sem_i0, sem_i1)
        sem_o = (sem_o0, sem_o1)

        def idx_copy(i, b):
            return pltpu.make_async_copy(
                idx_hbm.at[pl.ds(base0 + i * _CHUNK, _CHUNK)],
                idx_v[b], sem_i[b])

        def out_copy(i, b):
            return pltpu.make_async_copy(
                rows_v[b],
                out_hbm.at[pl.ds(base0 + i * _CHUNK, _CHUNK)],
                sem_o[b])

        pltpu.make_async_copy(table_hbm, table_v, sem_t).start()
        idx_copy(0, 0).start()
        idx_copy(1, 1).start()
        pltpu.make_async_copy(table_hbm, table_v, sem_t).wait()

        io = lax.iota(jnp.int32, _L)

        @pl.loop(0, n_chunks, step=_NBUF)
        def step(gi):
            for b in range(_NBUF):
                i = gi + b
                @pl.when(i >= _NBUF)
                def _():
                    out_copy(i - _NBUF, b).wait()

                idx_copy(i, b).wait()

                @pl.loop(0, n_groups)
                def grp(g):
                    i0 = g * _L
                    idx16 = idx_v[b][pl.ds(i0, _L)]
                    la = idx16 * d
                    rv = i0 + io
                    @pl.loop(0, d, unroll=8)
                    def colstep(j):
                        col = (io + j) & (d - 1)
                        v = plsc.load_gather(table_v, [la + col])
                        plsc.store_scatter(rows_v[b], [rv, col], v)

                out_copy(i, b).start()

                @pl.when(i + _NBUF < n_chunks)
                def _():
                    idx_copy(i + _NBUF, b).start()

        out_copy(n_chunks - 2, (n_chunks - 2) % _NBUF).wait()
        out_copy(n_chunks - 1, (n_chunks - 1) % _NBUF).wait()

    return k(table.reshape(-1), idx_flat)


def kernel(x, table):
    s0, s1 = x.shape
    d = table.shape[1]
    idx_flat = x.reshape(-1).astype(jnp.int32)
    out = _sc_embed(table, idx_flat, table.shape[0], d)
    return out.reshape(s0, s1, d)


# speedup vs baseline: 1.0660x; 1.0660x over previous
import functools

import jax
import jax.numpy as jnp
from jax import lax
from jax.experimental import pallas as pl
from jax.experimental.pallas import tpu as pltpu
from jax.experimental.pallas import tpu_sc as plsc

_NW = 32
_CHUNK = 400
_NBUF = 2
_L = 16


@functools.partial(jax.jit, static_argnames=("n_rows", "d"))
def _sc_embed(table, idx_flat, n_rows, d):
    b_total = idx_flat.shape[0]
    b_per_w = b_total // _NW
    n_chunks = b_per_w // _CHUNK
    mesh = plsc.VectorSubcoreMesh(core_axis_name="c", subcore_axis_name="s")

    @functools.partial(
        pl.kernel,
        mesh=mesh,
        out_type=jax.ShapeDtypeStruct((b_total, d), jnp.float32),
        scratch_types=[
            pltpu.VMEM((n_rows * d,), jnp.float32),
            pltpu.VMEM((_CHUNK,), jnp.int32),
            pltpu.VMEM((_CHUNK,), jnp.int32),
            pltpu.VMEM((_CHUNK, d), jnp.float32),
            pltpu.VMEM((_CHUNK, d), jnp.float32),
            pltpu.SemaphoreType.DMA,
            pltpu.SemaphoreType.DMA,
            pltpu.SemaphoreType.DMA,
            pltpu.SemaphoreType.DMA,
            pltpu.SemaphoreType.DMA,
        ],
        compiler_params=pltpu.CompilerParams(needs_layout_passes=False),
    )
    def k(table_hbm, idx_hbm, out_hbm, table_v, idx_v0, idx_v1,
          rows_v0, rows_v1, sem_t, sem_i0, sem_i1, sem_o0, sem_o1):
        cid = lax.axis_index("c")
        sid = lax.axis_index("s")
        wid = sid * 2 + cid
        base0 = wid * b_per_w
        idx_v = (idx_v0, idx_v1)
        rows_v = (rows_v0, rows_v1)
        sem_i = (sem_i0, sem_i1)
        sem_o = (sem_o0, sem_o1)

        def idx_copy(i, b):
            return pltpu.make_async_copy(
                idx_hbm.at[pl.ds(base0 + i * _CHUNK, _CHUNK)],
                idx_v[b], sem_i[b])

        def out_copy(i, b):
            return pltpu.make_async_copy(
                rows_v[b],
                out_hbm.at[pl.ds(base0 + i * _CHUNK, _CHUNK)],
                sem_o[b])

        pltpu.make_async_copy(table_hbm, table_v, sem_t).start()
        idx_copy(0, 0).start()
        idx_copy(1, 1).start()
        pltpu.make_async_copy(table_hbm, table_v, sem_t).wait()

        @pl.loop(0, n_chunks, step=_NBUF)
        def step(gi):
            for b in range(_NBUF):
                i = gi + b
                @pl.when(i >= _NBUF)
                def _():
                    out_copy(i - _NBUF, b).wait()

                idx_copy(i, b).wait()

                @pl.loop(0, _CHUNK // _L)
                def grp(g):
                    i0 = g * _L
                    la = idx_v[b][pl.ds(i0, _L)] * d
                    for l in range(_L):
                        s = la[l]
                        for cb in range(d // _L):
                            rows_v[b][i0 + l, pl.ds(cb * _L, _L)] = (
                                table_v[pl.ds(s + cb * _L, _L)])

                out_copy(i, b).start()

                @pl.when(i + _NBUF < n_chunks)
                def _():
                    idx_copy(i + _NBUF, b).start()

        out_copy(n_chunks - 2, (n_chunks - 2) % _NBUF).wait()
        out_copy(n_chunks - 1, (n_chunks - 1) % _NBUF).wait()

    return k(table.reshape(-1), idx_flat)


def kernel(x, table):
    s0, s1 = x.shape
    d = table.shape[1]
    idx_flat = x.reshape(-1).astype(jnp.int32)
    out = _sc_embed(table, idx_flat, table.shape[0], d)
    return out.reshape(s0, s1, d)
